# baseline (speedup 1.0000x reference)
import functools

import jax
import jax.numpy as jnp
from jax import lax
from jax.experimental import pallas as pl
from jax.experimental.pallas import tpu as pltpu
from jax.experimental.pallas import tpu_sc as plsc

_TILE = 256
_BIG = 1e30


def _body_a(posT_ref, q_ref, xp_ref, lf_ref, out_i_ref, out_w_ref, xr_ref,
            *, C):
    @pl.when(pl.program_id(0) == 0)
    def _():
        xpv = xp_ref[:]
        lfv = lf_ref[:]
        for k in range(3):
            acc = xpv[:, 0:C] * lfv[:, k:k + 1]
            acc += xpv[:, C:2 * C] * lfv[:, 3 + k:4 + k]
            acc += xpv[:, 2 * C:3 * C] * lfv[:, 6 + k:7 + k]
            xr_ref[:, k * C:(k + 1) * C] = acc
        xr_ref[:, 3 * C:] = jnp.zeros_like(xr_ref[:, 3 * C:])

    q = q_ref[:]
    kT = posT_ref[:]
    pp = jnp.dot(q, kT, preferred_element_type=jnp.float32)
    qx, qy, qz = q[:, 0:1], q[:, 1:2], q[:, 2:3]
    kx, ky, kz = kT[0:1], kT[1:2], kT[2:3]
    qn = (qx * qx + qz * qz) + qy * qy
    kn = (kx * kx + kz * kz) + ky * ky

    T = pp.shape[0]
    v1 = jnp.full((T, 128), _BIG, jnp.float32)
    v2 = v1
    v3 = v1
    j1 = jnp.zeros((T, 128), jnp.float32)
    j2 = j1
    j3 = j1
    for j in range(pp.shape[1] // 128):
        base = qn + kn[:, j * 128:(j + 1) * 128]
        dch = jnp.maximum(base - 2.0 * pp[:, j * 128:(j + 1) * 128], 0.0)
        jc = jnp.float32(j)
        c1 = dch < v1
        t = jnp.where(c1, v1, dch)
        jt = jnp.where(c1, j1, jc)
        v1 = jnp.where(c1, dch, v1)
        j1 = jnp.where(c1, jc, j1)
        c2 = t < v2
        u = jnp.where(c2, v2, t)
        ju = jnp.where(c2, j2, jt)
        v2 = jnp.where(c2, t, v2)
        j2 = jnp.where(c2, jt, j2)
        c3 = u < v3
        v3 = jnp.where(c3, u, v3)
        j3 = jnp.where(c3, ju, j3)
    lane = lax.broadcasted_iota(jnp.int32, (T, 128), 1).astype(jnp.float32)
    cand = jnp.concatenate([v1, v2, v3], axis=1)
    gidx = jnp.concatenate([j1 * 128.0 + lane,
                            j2 * 128.0 + lane,
                            j3 * 128.0 + lane], axis=1)
    sel_v = []
    sel_i = []
    for _r in range(3):
        v = jnp.min(cand, axis=1, keepdims=True)
        ii = jnp.min(jnp.where(cand == v, gidx, 4e9), axis=1, keepdims=True)
        cand = jnp.where(gidx == ii, _BIG, cand)
        sel_v.append(v)
        sel_i.append(ii)

    out_i_ref[:] = jnp.concatenate(sel_i, axis=1).astype(jnp.int32)
    out_w_ref[:] = jnp.concatenate(
        [1.0 / jnp.maximum(v, 1e-16) for v in sel_v], axis=1)


def _body_c(xg_ref, w3_ref, xs_ref, lfs_ref,
            W1a_ref, W1b_ref, W2_ref, b1_ref, b2_ref, out_ref, *, C):
    xg = xg_ref[:]
    w3 = w3_ref[:]
    w0, w1, w2 = w3[:, 0:1], w3[:, 1:2], w3[:, 2:3]
    s = (xg[:, 0:3 * C] * w0 + xg[:, 128:128 + 3 * C] * w1) \
        + xg[:, 256:256 + 3 * C] * w2
    den = (w0 + w1) + w2
    yp = s / den

    lfsv = lfs_ref[:]
    parts = []
    for i in range(3):
        acc = yp[:, 0:C] * lfsv[:, 3 * i:3 * i + 1]
        acc += yp[:, C:2 * C] * lfsv[:, 3 * i + 1:3 * i + 2]
        acc += yp[:, 2 * C:3 * C] * lfsv[:, 3 * i + 2:3 * i + 3]
        parts.append(acc)
    yr = jnp.concatenate(parts, axis=1)

    h = (jnp.dot(yr, W1a_ref[:], preferred_element_type=jnp.float32)
         + jnp.dot(xs_ref[:], W1b_ref[:], preferred_element_type=jnp.float32)
         + b1_ref[:])
    h = jnp.maximum(h, 0.0)
    out_ref[:] = (jnp.dot(h, W2_ref[:], preferred_element_type=jnp.float32)
                  + b2_ref[:])


def _make_sc_gather(V, B):
    info = plsc.get_sparse_core_info()
    n_workers = info.num_cores * info.num_subcores
    ch = 128
    b_per_w = B // n_workers
    n_ch = b_per_w // ch
    mesh = plsc.VectorSubcoreMesh(core_axis_name="c", subcore_axis_name="s")

    @functools.partial(
        pl.kernel, mesh=mesh,
        out_type=jax.ShapeDtypeStruct((B, 128), jnp.float32),
        scratch_types=[
            pltpu.VMEM((ch,), jnp.int32),
            pltpu.VMEM((ch, 128), jnp.float32),
            pltpu.SemaphoreType.DMA,
        ],
    )
    def sc_gather(table_hbm, idx_hbm, out_hbm, idx_v, rows_v, sem):
        wid = lax.axis_index("s") * info.num_cores + lax.axis_index("c")
        base = wid * b_per_w
        for c in range(n_ch):
            off = base + c * ch
            pltpu.sync_copy(idx_hbm.at[pl.ds(off, ch)], idx_v)
            pltpu.async_copy(table_hbm.at[idx_v], rows_v, sem).wait()
            pltpu.sync_copy(rows_v, out_hbm.at[pl.ds(off, ch)])

    return sc_gather


def kernel(x, pos, batch, lframes, x_skip, pos_skip, batch_skip, lframes_skip,
           W1, b1, W2, b2):
    del batch, batch_skip
    N, F = x.shape
    M = pos_skip.shape[0]
    C = F // 3
    H = W1.shape[1]

    xp = x.reshape(N, C, 3).transpose(0, 2, 1).reshape(N, F)
    lf = lframes.reshape(N, 9)
    lfs = lframes_skip.reshape(M, 9)
    posT = jnp.zeros((8, N), x.dtype).at[:3, :].set(pos.T)
    q = jnp.zeros((M, 8), x.dtype).at[:, :3].set(pos_skip)
    W1a = W1[:F].reshape(C, 3, H).transpose(1, 0, 2).reshape(F, H)
    W1b = W1[F:]
    b1r = b1.reshape(1, H)
    b2r = b2.reshape(1, H)

    full = lambda s: pl.BlockSpec(s, lambda i: (0, 0))
    tiled = lambda s: pl.BlockSpec(s, lambda i: (i, 0))

    def stage_a(qh, Mh):
        return pl.pallas_call(
            functools.partial(_body_a, C=C),
            grid=(Mh // _TILE,),
            in_specs=[
                full((8, N)),
                tiled((_TILE, 8)),
                full((N, F)),
                full((N, 9)),
            ],
            out_specs=[tiled((_TILE, 3)),
                       tiled((_TILE, 3)),
                       full((N, 128))],
            out_shape=[jax.ShapeDtypeStruct((Mh, 3), jnp.int32),
                       jax.ShapeDtypeStruct((Mh, 3), jnp.float32),
                       jax.ShapeDtypeStruct((N, 128), jnp.float32)],
        )(posT, qh, xp, lf)

    def stage_c(xgh, w3h, xsh, lfsh, Mh):
        return pl.pallas_call(
            functools.partial(_body_c, C=C),
            grid=(Mh // _TILE,),
            in_specs=[
                tiled((_TILE, 3 * 128)),
                tiled((_TILE, 3)),
                tiled((_TILE, F)),
                tiled((_TILE, 9)),
                full((F, H)),
                full((F, H)),
                full((H, H)),
                full((1, H)),
                full((1, H)),
            ],
            out_specs=tiled((_TILE, H)),
            out_shape=jax.ShapeDtypeStruct((Mh, H), x.dtype),
        )(xgh.reshape(Mh, 3 * 128), w3h, xsh, lfsh, W1a, W1b, W2, b1r, b2r)

    Mh = M // 2
    gather = _make_sc_gather(N, Mh * 3)
    i1, w1, xr_p = stage_a(q[:Mh], Mh)
    i2, w2, _ = stage_a(q[Mh:], Mh)
    xg1 = gather(xr_p, i1.reshape(Mh * 3))
    xg2 = gather(xr_p, i2.reshape(Mh * 3))
    out1 = stage_c(xg1, w1, x_skip[:Mh], lfs[:Mh], Mh)
    out2 = stage_c(xg2, w2, x_skip[Mh:], lfs[Mh:], Mh)
    return jnp.concatenate([out1, out2], axis=0)


# speedup vs baseline: 1.0563x; 1.0563x over previous
import functools

import jax
import jax.numpy as jnp
from jax import lax
from jax.experimental import pallas as pl
from jax.experimental.pallas import tpu as pltpu
from jax.experimental.pallas import tpu_sc as plsc

_TILE = 256
_BIG = 1e30


def _body_a(posT_ref, q_ref, xp_ref, lf_ref, out_i_ref, out_w_ref, xr_ref,
            *, C):
    @pl.when(pl.program_id(0) == 0)
    def _():
        xpv = xp_ref[:]
        lfv = lf_ref[:]
        for k in range(3):
            acc = xpv[:, 0:C] * lfv[:, k:k + 1]
            acc += xpv[:, C:2 * C] * lfv[:, 3 + k:4 + k]
            acc += xpv[:, 2 * C:3 * C] * lfv[:, 6 + k:7 + k]
            xr_ref[:, k * C:(k + 1) * C] = acc
        xr_ref[:, 3 * C:] = jnp.zeros_like(xr_ref[:, 3 * C:])

    q = q_ref[:]
    kT = posT_ref[:]
    pp = jnp.dot(q, kT, preferred_element_type=jnp.float32)
    qx, qy, qz = q[:, 0:1], q[:, 1:2], q[:, 2:3]
    kx, ky, kz = kT[0:1], kT[1:2], kT[2:3]
    qn = (qx * qx + qz * qz) + qy * qy
    kn = (kx * kx + kz * kz) + ky * ky

    T = pp.shape[0]
    v1 = jnp.full((T, 128), _BIG, jnp.float32)
    v2 = v1
    v3 = v1
    j1 = jnp.zeros((T, 128), jnp.float32)
    j2 = j1
    j3 = j1
    for j in range(pp.shape[1] // 128):
        base = qn + kn[:, j * 128:(j + 1) * 128]
        dch = jnp.maximum(base - 2.0 * pp[:, j * 128:(j + 1) * 128], 0.0)
        jc = jnp.float32(j)
        c1 = dch < v1
        t = jnp.where(c1, v1, dch)
        jt = jnp.where(c1, j1, jc)
        v1 = jnp.where(c1, dch, v1)
        j1 = jnp.where(c1, jc, j1)
        c2 = t < v2
        u = jnp.where(c2, v2, t)
        ju = jnp.where(c2, j2, jt)
        v2 = jnp.where(c2, t, v2)
        j2 = jnp.where(c2, jt, j2)
        c3 = u < v3
        v3 = jnp.where(c3, u, v3)
        j3 = jnp.where(c3, ju, j3)
    lane = lax.broadcasted_iota(jnp.int32, (T, 128), 1).astype(jnp.float32)
    cand = jnp.concatenate([v1, v2, v3], axis=1)
    gidx = jnp.concatenate([j1 * 128.0 + lane,
                            j2 * 128.0 + lane,
                            j3 * 128.0 + lane], axis=1)
    sel_v = []
    sel_i = []
    for _r in range(3):
        v = jnp.min(cand, axis=1, keepdims=True)
        ii = jnp.min(jnp.where(cand == v, gidx, 4e9), axis=1, keepdims=True)
        cand = jnp.where(gidx == ii, _BIG, cand)
        sel_v.append(v)
        sel_i.append(ii)

    out_i_ref[:] = jnp.concatenate(sel_i, axis=1).astype(jnp.int32)
    out_w_ref[:] = jnp.concatenate(
        [1.0 / jnp.maximum(v, 1e-16) for v in sel_v], axis=1)


def _body_c(xg_ref, w3_ref, xs_ref, lfs_ref,
            W1a_ref, W1b_ref, W2_ref, b1_ref, b2_ref, out_ref, *, C):
    xg = xg_ref[:]
    w3 = w3_ref[:]
    w0, w1, w2 = w3[:, 0:1], w3[:, 1:2], w3[:, 2:3]
    s = (xg[:, 0:3 * C] * w0 + xg[:, 128:128 + 3 * C] * w1) \
        + xg[:, 256:256 + 3 * C] * w2
    den = (w0 + w1) + w2
    yp = s / den

    lfsv = lfs_ref[:]
    parts = []
    for i in range(3):
        acc = yp[:, 0:C] * lfsv[:, 3 * i:3 * i + 1]
        acc += yp[:, C:2 * C] * lfsv[:, 3 * i + 1:3 * i + 2]
        acc += yp[:, 2 * C:3 * C] * lfsv[:, 3 * i + 2:3 * i + 3]
        parts.append(acc)
    yr = jnp.concatenate(parts, axis=1)

    h = (jnp.dot(yr, W1a_ref[:], preferred_element_type=jnp.float32)
         + jnp.dot(xs_ref[:], W1b_ref[:], preferred_element_type=jnp.float32)
         + b1_ref[:])
    h = jnp.maximum(h, 0.0)
    out_ref[:] = (jnp.dot(h, W2_ref[:], preferred_element_type=jnp.float32)
                  + b2_ref[:])


def _make_sc_gather(V, B):
    info = plsc.get_sparse_core_info()
    n_workers = info.num_cores * info.num_subcores
    ch = 128
    b_per_w = B // n_workers
    n_ch = b_per_w // ch
    mesh = plsc.VectorSubcoreMesh(core_axis_name="c", subcore_axis_name="s")

    @functools.partial(
        pl.kernel, mesh=mesh,
        out_type=jax.ShapeDtypeStruct((B, 128), jnp.float32),
        scratch_types=[
            pltpu.VMEM((ch,), jnp.int32),
            pltpu.VMEM((ch, 128), jnp.float32),
            pltpu.SemaphoreType.DMA,
        ],
    )
    def sc_gather(table_hbm, idx_hbm, out_hbm, idx_v, rows_v, sem):
        wid = lax.axis_index("s") * info.num_cores + lax.axis_index("c")
        base = wid * b_per_w
        for c in range(n_ch):
            off = base + c * ch
            pltpu.sync_copy(idx_hbm.at[pl.ds(off, ch)], idx_v)
            pltpu.async_copy(table_hbm.at[idx_v], rows_v, sem).wait()
            pltpu.sync_copy(rows_v, out_hbm.at[pl.ds(off, ch)])

    return sc_gather


def kernel(x, pos, batch, lframes, x_skip, pos_skip, batch_skip, lframes_skip,
           W1, b1, W2, b2):
    del batch, batch_skip
    N, F = x.shape
    M = pos_skip.shape[0]
    C = F // 3
    H = W1.shape[1]

    xp = x.reshape(N, C, 3).transpose(0, 2, 1).reshape(N, F)
    lf = lframes.reshape(N, 9)
    lfs = lframes_skip.reshape(M, 9)
    posT = jnp.zeros((8, N), x.dtype).at[:3, :].set(pos.T)
    q = jnp.zeros((M, 8), x.dtype).at[:, :3].set(pos_skip)
    W1a = W1[:F].reshape(C, 3, H).transpose(1, 0, 2).reshape(F, H)
    W1b = W1[F:]
    b1r = b1.reshape(1, H)
    b2r = b2.reshape(1, H)

    full = lambda s: pl.BlockSpec(s, lambda i: (0, 0))
    tiled = lambda s: pl.BlockSpec(s, lambda i: (i, 0))

    def stage_a(qh, Mh):
        return pl.pallas_call(
            functools.partial(_body_a, C=C),
            grid=(Mh // _TILE,),
            in_specs=[
                full((8, N)),
                tiled((_TILE, 8)),
                full((N, F)),
                full((N, 9)),
            ],
            out_specs=[tiled((_TILE, 3)),
                       tiled((_TILE, 3)),
                       full((N, 128))],
            out_shape=[jax.ShapeDtypeStruct((Mh, 3), jnp.int32),
                       jax.ShapeDtypeStruct((Mh, 3), jnp.float32),
                       jax.ShapeDtypeStruct((N, 128), jnp.float32)],
        )(posT, qh, xp, lf)

    def stage_c(xgh, w3h, xsh, lfsh, Mh):
        return pl.pallas_call(
            functools.partial(_body_c, C=C),
            grid=(Mh // _TILE,),
            in_specs=[
                tiled((_TILE, 3 * 128)),
                tiled((_TILE, 3)),
                tiled((_TILE, F)),
                tiled((_TILE, 9)),
                full((F, H)),
                full((F, H)),
                full((H, H)),
                full((1, H)),
                full((1, H)),
            ],
            out_specs=tiled((_TILE, H)),
            out_shape=jax.ShapeDtypeStruct((Mh, H), x.dtype),
        )(xgh.reshape(Mh, 3 * 128), w3h, xsh, lfsh, W1a, W1b, W2, b1r, b2r)

    idx3, w3, xr_p = stage_a(q, M)
    xg = _make_sc_gather(N, M * 3)(xr_p, idx3.reshape(M * 3))
    return stage_c(xg, w3, x_skip, lfs, M)
